# baseline (speedup 1.0000x reference)
import functools

import jax
import jax.numpy as jnp
from jax import lax
from jax.experimental import pallas as pl
from jax.experimental.pallas import tpu as pltpu
from jax.experimental.pallas import tpu_sc as plsc


B, N, M = 8, 2048, 2048
NC, NS, L = 2, 16, 16
NW = NC * NS
CPB = NW // B
QCH = N // CPB
NG = QCH // L
NBLK = N // L

_mesh = plsc.VectorSubcoreMesh(core_axis_name="c", subcore_axis_name="s")

_GDN = lax.GatherDimensionNumbers(
    offset_dims=(), collapsed_slice_dims=(0,), start_index_map=(0,))


def _perm(v, idx):
    return lax.gather(v, idx[:, None], dimension_numbers=_GDN,
                      slice_sizes=(1,),
                      mode=lax.GatherScatterMode.PROMISE_IN_BOUNDS)


def _xlane_reduce(v, op):
    lane = lax.iota(jnp.int32, L)
    for sh in (8, 4, 2, 1):
        v = op(v, _perm(v, lane ^ sh))
    return v[0]


def _xlane_add(v):
    return _xlane_reduce(v, jnp.add)


def _scan_block(best, qxv, qyv, qzv, cx, cy, cz, blk, rots):
    o = blk * L
    px = cx[pl.ds(o, L)]
    py = cy[pl.ds(o, L)]
    pz = cz[pl.ds(o, L)]
    for r in range(L):
        if r == 0:
            pxr, pyr, pzr = px, py, pz
        else:
            pxr = _perm(px, rots[r - 1])
            pyr = _perm(py, rots[r - 1])
            pzr = _perm(pz, rots[r - 1])
        dx = qxv - pxr
        dy = qyv - pyr
        dz = qzv - pzr
        d2 = dx * dx + dy * dy + dz * dz
        best = jnp.minimum(best, d2)
    return best


CAP = 12


def _pass(qx, qy, qz, cx, cy, cz, bmin_v, bmax_v, chunk, rot1):
    gbase = chunk * NG

    def group_body(g, acc):
        gg = gbase + g
        q0 = gg * L
        qxv = qx[pl.ds(q0, L)]
        qyv = qy[pl.ds(q0, L)]
        qzv = qz[pl.ds(q0, L)]
        qlo = _xlane_reduce(qxv, jnp.minimum)
        qhi = _xlane_reduce(qxv, jnp.maximum)
        best = _scan_block(jnp.full((L,), jnp.inf, jnp.float32),
                           qxv, qyv, qzv, cx, cy, cz, gg, rot1)
        ubm = _xlane_reduce(best, jnp.maximum)

        one = jnp.full((L,), 1.0, jnp.float32)
        zero = jnp.zeros((L,), jnp.float32)

        def win_body(i, cnt):
            cl, cr = cnt
            bm = bmax_v[pl.ds(i * L, L)]
            e = qlo - bm
            cl = cl + jnp.where(jnp.logical_and(e > 0, e * e > ubm), one,
                                zero)
            bn = bmin_v[pl.ds(i * L, L)]
            e2 = bn - qhi
            cr = cr + jnp.where(jnp.logical_and(e2 > 0, e2 * e2 > ubm), one,
                                zero)
            return cl, cr

        cl, cr = lax.fori_loop(0, NBLK // L, win_body, (zero, zero))
        lo1 = _xlane_add(cl).astype(jnp.int32)
        hi1 = jnp.maximum(NBLK - _xlane_add(cr).astype(jnp.int32), lo1)

        def blk_body(blk, b):
            return _scan_block(b, qxv, qyv, qzv, cx, cy, cz, blk, rot1)

        sl = jnp.maximum(lo1, gg - CAP)
        sh = jnp.maximum(jnp.minimum(hi1, gg + CAP + 1), sl)
        best = plsc.parallel_loop(sl, sh, carry=best)(blk_body)

        ubm = _xlane_reduce(best, jnp.maximum)
        cl, cr = lax.fori_loop(0, NBLK // L, win_body, (zero, zero))
        lo2 = _xlane_add(cl).astype(jnp.int32)
        hi2 = jnp.maximum(NBLK - _xlane_add(cr).astype(jnp.int32), lo2)
        end_l = jnp.maximum(jnp.minimum(sl, hi2), lo2)
        beg_r = jnp.maximum(sh, lo2)
        end_r = jnp.maximum(hi2, beg_r)
        nleft = end_l - lo2

        def seg_body(j, b):
            blk = jnp.where(j < nleft, lo2 + j, beg_r + (j - nleft))
            return _scan_block(b, qxv, qyv, qzv, cx, cy, cz, blk, rot1)

        best = plsc.parallel_loop(
            jnp.int32(0), nleft + (end_r - beg_r), carry=best)(seg_body)
        return acc + best

    acc = lax.fori_loop(0, NG, group_body, jnp.zeros((L,), jnp.float32))
    return _xlane_add(acc)


@functools.partial(
    pl.kernel,
    out_type=jax.ShapeDtypeStruct((NW * L,), jnp.float32),
    mesh=_mesh,
    scratch_types=[
        pltpu.VMEM((N,), jnp.float32),
        pltpu.VMEM((N,), jnp.float32),
        pltpu.VMEM((N,), jnp.float32),
        pltpu.VMEM((M,), jnp.float32),
        pltpu.VMEM((M,), jnp.float32),
        pltpu.VMEM((M,), jnp.float32),
        pltpu.VMEM((NBLK,), jnp.float32),
        pltpu.VMEM((NBLK,), jnp.float32),
        pltpu.VMEM((NBLK,), jnp.float32),
        pltpu.VMEM((NBLK,), jnp.float32),
        pltpu.VMEM((L,), jnp.float32),
    ],
)
def _stage1(coords_h, edges_h, sum_h,
            ax, ay, az, bx, by, bz,
            gbmin, gbmax, pbmin, pbmax, rs_v):
    wid = lax.axis_index("c") * NS + lax.axis_index("s")
    b = wid // CPB
    chunk = wid % CPB

    pltpu.sync_copy(coords_h.at[pl.ds((0 * B + b) * N, N)], ax)
    pltpu.sync_copy(coords_h.at[pl.ds((1 * B + b) * N, N)], ay)
    pltpu.sync_copy(coords_h.at[pl.ds((2 * B + b) * N, N)], az)
    pltpu.sync_copy(coords_h.at[pl.ds((3 * B + b) * N, N)], bx)
    pltpu.sync_copy(coords_h.at[pl.ds((4 * B + b) * N, N)], by)
    pltpu.sync_copy(coords_h.at[pl.ds((5 * B + b) * N, N)], bz)
    pltpu.sync_copy(edges_h.at[pl.ds((0 * B + b) * NBLK, NBLK)], gbmin)
    pltpu.sync_copy(edges_h.at[pl.ds((1 * B + b) * NBLK, NBLK)], gbmax)
    pltpu.sync_copy(edges_h.at[pl.ds((2 * B + b) * NBLK, NBLK)], pbmin)
    pltpu.sync_copy(edges_h.at[pl.ds((3 * B + b) * NBLK, NBLK)], pbmax)

    rot1 = tuple((lax.iota(jnp.int32, L) + r) & (L - 1)
                 for r in range(1, L))

    total = _pass(ax, ay, az, bx, by, bz, pbmin, pbmax, chunk, rot1)
    total = total + _pass(bx, by, bz, ax, ay, az, gbmin, gbmax, chunk, rot1)

    lane = lax.iota(jnp.int32, L)
    rs_v[...] = jnp.where(lane == 0, total, jnp.float32(0))
    pltpu.sync_copy(rs_v, sum_h.at[pl.ds(wid * L, L)])


@functools.partial(
    pl.kernel,
    out_type=jax.ShapeDtypeStruct((L,), jnp.float32),
    mesh=_mesh,
    scratch_types=[
        pltpu.VMEM((NW * L,), jnp.float32),
        pltpu.VMEM((L,), jnp.float32),
    ],
)
def _stage2(sum_h, out_h, rs_v, o_v):
    wid = lax.axis_index("c") * NS + lax.axis_index("s")

    @pl.when(wid == 0)
    def _():
        pltpu.sync_copy(sum_h, rs_v)

        def row_body(w, acc):
            return acc + rs_v[pl.ds(w * L, L)]

        acc = lax.fori_loop(0, NW, row_body, jnp.zeros((L,), jnp.float32))
        total = _xlane_add(acc) * jnp.float32(1.0 / (B * N))
        lane = lax.iota(jnp.int32, L)
        o_v[...] = jnp.where(lane == 0, total, jnp.float32(0))
        pltpu.sync_copy(o_v, out_h)


def kernel(gt, pred):
    gx, gy, gz = gt[:, :, 0], gt[:, :, 1], gt[:, :, 2]
    px, py, pz = pred[:, :, 0], pred[:, :, 1], pred[:, :, 2]
    go = jnp.argsort(gx, axis=1)
    po = jnp.argsort(px, axis=1)
    gxs = jnp.take_along_axis(gx, go, axis=1)
    gys = jnp.take_along_axis(gy, go, axis=1)
    gzs = jnp.take_along_axis(gz, go, axis=1)
    pxs = jnp.take_along_axis(px, po, axis=1)
    pys = jnp.take_along_axis(py, po, axis=1)
    pzs = jnp.take_along_axis(pz, po, axis=1)
    coords = jnp.stack([gxs, gys, gzs, pxs, pys, pzs]).reshape(-1)
    edges = jnp.stack([gxs[:, ::L], gxs[:, L - 1::L],
                       pxs[:, ::L], pxs[:, L - 1::L]]).reshape(-1)
    sums = _stage1(coords, edges)
    out = _stage2(sums)
    return out[0]


# speedup vs baseline: 2.1497x; 2.1497x over previous
import functools

import jax
import jax.numpy as jnp
from jax import lax
from jax.experimental import pallas as pl
from jax.experimental.pallas import tpu as pltpu
from jax.experimental.pallas import tpu_sc as plsc


B, N, M = 8, 2048, 2048
NC, NS, L = 2, 16, 16
NW = NC * NS
GPB = NW // B
CH = N // GPB
JB = M // L
ITILE = 8

_mesh = plsc.VectorSubcoreMesh(core_axis_name="c", subcore_axis_name="s")

_GDN = lax.GatherDimensionNumbers(
    offset_dims=(), collapsed_slice_dims=(0,), start_index_map=(0,))


def _perm(v, idx):
    return lax.gather(v, idx[:, None], dimension_numbers=_GDN,
                      slice_sizes=(1,),
                      mode=lax.GatherScatterMode.PROMISE_IN_BOUNDS)


def _xlane_reduce(v, op):
    lane = lax.iota(jnp.int32, L)
    for sh in (8, 4, 2, 1):
        v = op(v, _perm(v, lane ^ sh))
    return v[0]


@functools.partial(
    pl.kernel,
    out_type=[
        jax.ShapeDtypeStruct((NW, M), jnp.float32),
        jax.ShapeDtypeStruct((NW, L), jnp.float32),
    ],
    mesh=_mesh,
    scratch_types=[
        pltpu.VMEM((CH,), jnp.float32),
        pltpu.VMEM((CH,), jnp.float32),
        pltpu.VMEM((CH,), jnp.float32),
        pltpu.VMEM((M,), jnp.float32),
        pltpu.VMEM((M,), jnp.float32),
        pltpu.VMEM((M,), jnp.float32),
        pltpu.VMEM((M,), jnp.float32),
        pltpu.VMEM((L,), jnp.float32),
    ],
)
def _stage1(gx_h, gy_h, gz_h, px_h, py_h, pz_h, colmin_h, rowsum_h,
            gx, gy, gz, px, py, pz, colmin, rs_v):
    wid = lax.axis_index("c") * NS + lax.axis_index("s")
    b = wid // GPB
    chunk = wid % GPB
    g0 = chunk * CH

    pltpu.sync_copy(gx_h.at[b, pl.ds(g0, CH)], gx)
    pltpu.sync_copy(gy_h.at[b, pl.ds(g0, CH)], gy)
    pltpu.sync_copy(gz_h.at[b, pl.ds(g0, CH)], gz)
    pltpu.sync_copy(px_h.at[b], px)
    pltpu.sync_copy(py_h.at[b], py)
    pltpu.sync_copy(pz_h.at[b], pz)

    inf_v = jnp.full((L,), jnp.inf, jnp.float32)

    def init_body(j, carry):
        colmin[pl.ds(j * L, L)] = inf_v
        return carry

    lax.fori_loop(0, JB, init_body, jnp.int32(0))

    def group_body(it, rowsum):
        base = it * L
        gxv = gx[pl.ds(base, L)]
        gyv = gy[pl.ds(base, L)]
        gzv = gz[pl.ds(base, L)]
        for kk in range(L // ITILE):
            gs = [(gxv[kk * ITILE + k], gyv[kk * ITILE + k],
                   gzv[kk * ITILE + k]) for k in range(ITILE)]

            def jb_body(j, rms):
                o = j * L
                pxv = px[pl.ds(o, L)]
                pyv = py[pl.ds(o, L)]
                pzv = pz[pl.ds(o, L)]
                cm = colmin[pl.ds(o, L)]
                new_rms = []
                for k in range(ITILE):
                    gxk, gyk, gzk = gs[k]
                    dx = pxv - gxk
                    dy = pyv - gyk
                    dz = pzv - gzk
                    d2 = dx * dx + dy * dy + dz * dz
                    new_rms.append(jnp.minimum(rms[k], d2))
                    cm = jnp.minimum(cm, d2)
                colmin[pl.ds(o, L)] = cm
                return tuple(new_rms)

            rms = lax.fori_loop(0, JB, jb_body,
                                tuple(inf_v for _ in range(ITILE)))
            for k in range(ITILE):
                rowsum = rowsum + _xlane_reduce(rms[k], jnp.minimum)
        return rowsum

    rowsum = lax.fori_loop(0, CH // L, group_body, jnp.float32(0))

    pltpu.sync_copy(colmin, colmin_h.at[wid])
    lane = lax.iota(jnp.int32, L)
    rs_v[...] = jnp.where(lane == 0, rowsum, jnp.float32(0))
    pltpu.sync_copy(rs_v, rowsum_h.at[wid])


@functools.partial(
    pl.kernel,
    out_type=jax.ShapeDtypeStruct((L,), jnp.float32),
    mesh=_mesh,
    scratch_types=[
        pltpu.VMEM((NW, M), jnp.float32),
        pltpu.VMEM((NW, L), jnp.float32),
        pltpu.VMEM((L,), jnp.float32),
    ],
)
def _stage2(colmin_h, rowsum_h, out_h, cm_v, rs_v, o_v):
    wid = lax.axis_index("c") * NS + lax.axis_index("s")

    @pl.when(wid == 0)
    def _():
        pltpu.sync_copy(colmin_h, cm_v)
        pltpu.sync_copy(rowsum_h, rs_v)

        def col_body(t, acc):
            bb = t // JB
            j = t % JB
            o = j * L
            w0 = bb * GPB
            m = cm_v[w0, pl.ds(o, L)]
            m = jnp.minimum(m, cm_v[w0 + 1, pl.ds(o, L)])
            m = jnp.minimum(m, cm_v[w0 + 2, pl.ds(o, L)])
            m = jnp.minimum(m, cm_v[w0 + 3, pl.ds(o, L)])
            return acc + m

        col_acc = lax.fori_loop(0, B * JB, col_body,
                                jnp.zeros((L,), jnp.float32))

        def row_body(w, acc):
            return acc + rs_v[w]

        row_acc = lax.fori_loop(0, NW, row_body, jnp.zeros((L,), jnp.float32))

        total = (_xlane_reduce(col_acc, jnp.add) +
                 _xlane_reduce(row_acc, jnp.add)) * jnp.float32(1.0 / (B * N))
        lane = lax.iota(jnp.int32, L)
        o_v[...] = jnp.where(lane == 0, total, jnp.float32(0))
        pltpu.sync_copy(o_v, out_h)


def kernel(gt, pred):
    gx = jnp.asarray(gt[:, :, 0])
    gy = jnp.asarray(gt[:, :, 1])
    gz = jnp.asarray(gt[:, :, 2])
    px = jnp.asarray(pred[:, :, 0])
    py = jnp.asarray(pred[:, :, 1])
    pz = jnp.asarray(pred[:, :, 2])
    colmin, rowsum = _stage1(gx, gy, gz, px, py, pz)
    out = _stage2(colmin, rowsum)
    return out[0]
